# Initial kernel scaffold; baseline (speedup 1.0000x reference)
import jax
import jax.numpy as jnp
from jax.experimental import pallas as pl


def kernel(x, W):
    raise NotImplementedError("write your pallas kernel here")


# speedup vs baseline: 1.7849x; 1.7849x over previous
import jax
import jax.numpy as jnp
from jax.experimental import pallas as pl

_CAP_FRAC = 0.5
_AUX_W = 0.01


def _matvec_body(x_ref, w_ref, out_ref):
    out_ref[...] = jax.lax.dot_general(
        x_ref[...].astype(jnp.bfloat16), w_ref[...].astype(jnp.bfloat16),
        dimension_numbers=(((1,), (0,)), ((), ())),
        preferred_element_type=jnp.float32,
    )


def _compute_logits(x2, w2, blk=1024):
    n, d = x2.shape
    return pl.pallas_call(
        _matvec_body,
        grid=(n // blk,),
        in_specs=[
            pl.BlockSpec((blk, d), lambda i: (i, 0)),
            pl.BlockSpec((d, 1), lambda i: (0, 0)),
        ],
        out_specs=pl.BlockSpec((blk, 1), lambda i: (i, 0)),
        out_shape=jax.ShapeDtypeStruct((n, 1), jnp.float32),
    )(x2, w2)


def _make_mask_body(cap):
    def _mask_body(logits_ref, noise_ref, mask_ref, aux_ref):
        lg = logits_ref[...]
        b_, l_ = lg.shape
        noisy = lg + noise_ref[...]
        ui = jax.lax.bitcast_convert_type(noisy, jnp.int32)
        ikey = jnp.where(ui < 0, ui ^ jnp.int32(0x7FFFFFFF), ui)
        msb = jnp.int32(-2147483648)

        def step(i, u):
            cand = u | jax.lax.shift_left(jnp.int32(1), jnp.int32(31) - i)
            cand_s = cand ^ msb
            cnt = jnp.sum((ikey >= cand_s).astype(jnp.int32), axis=1,
                          keepdims=True)
            return jnp.where(cnt >= cap, cand, u)

        u = jax.lax.fori_loop(0, 32, step, jnp.zeros((b_, 1), jnp.int32))
        t = u ^ msb
        gt = ikey > t
        eq = ikey == t
        n_gt = jnp.sum(gt.astype(jnp.int32), axis=1, keepdims=True)
        rem = cap - n_gt
        c = eq.astype(jnp.int32)
        sh = 1
        while sh < l_:
            c = c + jnp.concatenate(
                [jnp.zeros((b_, sh), jnp.int32), c[:, : l_ - sh]], axis=1)
            sh *= 2
        mask = gt | (eq & (c <= rem))
        mask_ref[...] = mask.astype(jnp.int32)

        probs = 1.0 / (1.0 + jnp.exp(-lg))
        rowmean = jnp.sum(probs, axis=1, keepdims=True) * (1.0 / l_)
        dev = rowmean - _CAP_FRAC
        aux_ref[...] = jnp.sum(dev * dev, axis=0, keepdims=True) * (_AUX_W / b_)

    return _mask_body


def _mask_and_aux(logits, noise, cap):
    b, l = logits.shape
    return pl.pallas_call(
        _make_mask_body(cap),
        in_specs=[
            pl.BlockSpec((b, l), lambda: (0, 0)),
            pl.BlockSpec((b, l), lambda: (0, 0)),
        ],
        out_specs=[
            pl.BlockSpec((b, l), lambda: (0, 0)),
            pl.BlockSpec((1, 1), lambda: (0, 0)),
        ],
        out_shape=[
            jax.ShapeDtypeStruct((b, l), jnp.int32),
            jax.ShapeDtypeStruct((1, 1), jnp.float32),
        ],
    )(logits, noise)


def kernel(x, W):
    b, l, d = x.shape
    cap = max(1, int(l * _CAP_FRAC))
    logits2 = _compute_logits(x.reshape(b * l, d), W.reshape(d, 1))
    logits = logits2.reshape(b, l)
    noise = jax.random.normal(jax.random.key(1), (b, l), jnp.float32) * 0.1
    mask_i, aux = _mask_and_aux(logits, noise, cap)
    return mask_i.astype(jnp.bool_), logits, aux[0, 0]
